# baseline (speedup 1.0000x reference)
import functools

import jax
import jax.numpy as jnp
from jax import lax
from jax.experimental import pallas as pl
from jax.experimental.pallas import tpu as pltpu
from jax.experimental.pallas import tpu_sc as plsc

_C = 1000
_RATIO = 3
_G_COEF = (0.9999999208709724, -0.4999925589798402, 0.333160106694737,
           -0.24825648584296323, 0.1905595564947489, -0.13583941890268855,
           0.07754038118242754, -0.029210267904388937, 0.00518600370045478)
_NW = 32
_SC_CH = 16


def _sc_contrib(xv, yv):
    u = jnp.exp(-jnp.abs(xv))
    g = jnp.float32(_G_COEF[8])
    for k in range(7, -1, -1):
        g = g * u + jnp.float32(_G_COEF[k])
    s = jnp.maximum(xv, 0.0) + u * g
    return s - yv * xv


def _make_sc_call(n_rows, base_offset):
    rpw = n_rows // _NW
    nchunks = rpw // _SC_CH
    mesh = plsc.VectorSubcoreMesh(core_axis_name="c", subcore_axis_name="s")

    @functools.partial(
        pl.kernel,
        mesh=mesh,
        out_type=[
            jax.ShapeDtypeStruct((_NW, 16), jnp.float32),
            jax.ShapeDtypeStruct((_NW, 16), jnp.float32),
            jax.ShapeDtypeStruct((_NW, 16), jnp.float32),
        ],
        scratch_types=[
            pltpu.VMEM((2, _SC_CH, _C), jnp.float32),
            pltpu.VMEM((2, _SC_CH, _C), jnp.float32),
            pltpu.VMEM((16,), jnp.float32),
            pltpu.SemaphoreType.DMA((2,)),
            pltpu.SemaphoreType.DMA((2,)),
        ],
        compiler_params=pltpu.CompilerParams(needs_layout_passes=False),
    )
    def sc_kernel(pred_hbm, target_hbm, onum, oden, ominp,
                  bufx, bufy, vout, semx, semy):
        wid = lax.axis_index("s") * 2 + lax.axis_index("c")
        base = base_offset + wid * rpw
        tailmask = lax.iota(jnp.int32, 16) < 8

        def issue(chunk, slot):
            row0 = base + chunk * _SC_CH
            pltpu.make_async_copy(
                pred_hbm.at[pl.ds(row0, _SC_CH), :], bufx.at[slot], semx.at[slot]
            ).start()
            pltpu.make_async_copy(
                target_hbm.at[pl.ds(row0, _SC_CH), :], bufy.at[slot], semy.at[slot]
            ).start()

        issue(0, 0)

        def chunk_body(ci, carry):
            accnum, den_v, minp_v = carry
            slot = lax.rem(ci, 2)

            @pl.when(ci + 1 < nchunks)
            def _issue_next():
                issue(ci + 1, 1 - slot)

            row0 = base + ci * _SC_CH
            pltpu.make_async_copy(
                pred_hbm.at[pl.ds(row0, _SC_CH), :], bufx.at[slot], semx.at[slot]
            ).wait()
            pltpu.make_async_copy(
                target_hbm.at[pl.ds(row0, _SC_CH), :], bufy.at[slot], semy.at[slot]
            ).wait()

            def row_body(r, carry2):
                accs, dv, mv = carry2
                accs = list(accs)
                poss = [jnp.zeros((16,), jnp.int32) for _ in range(4)]
                for k in range(62):
                    xv = bufx[slot, r, pl.ds(k * 16, 16)]
                    yv = bufy[slot, r, pl.ds(k * 16, 16)]
                    accs[k % 8] = accs[k % 8] + _sc_contrib(xv, yv)
                    poss[k % 4] = poss[k % 4] + plsc.all_reduce_population_count(yv > 0.5)
                xv = bufx[slot, r, pl.ds(984, 16)]
                yv = bufy[slot, r, pl.ds(984, 16)]
                xv = jnp.where(tailmask, jnp.float32(-1e30), xv)
                yv = jnp.where(tailmask, jnp.float32(0.0), yv)
                accs[6] = accs[6] + _sc_contrib(xv, yv)
                poss[2] = poss[2] + plsc.all_reduce_population_count(yv > 0.5)
                pos = (poss[0] + poss[1]) + (poss[2] + poss[3])
                posf = pos.astype(jnp.float32)
                return (tuple(accs), dv + posf, jnp.minimum(mv, posf))

            return lax.fori_loop(0, _SC_CH, row_body, (accnum, den_v, minp_v))

        acc0 = tuple(jnp.zeros((16,), jnp.float32) for _ in range(8))
        accs, den_v, minp_v = lax.fori_loop(
            0, nchunks, chunk_body,
            (acc0, jnp.zeros((16,), jnp.float32), jnp.full((16,), 1e9, jnp.float32)),
        )
        a01 = (accs[0] + accs[1]) + (accs[2] + accs[3])
        a23 = (accs[4] + accs[5]) + (accs[6] + accs[7])
        vout[...] = a01 + a23
        pltpu.sync_copy(vout, onum.at[wid])
        vout[...] = den_v
        pltpu.sync_copy(vout, oden.at[wid])
        vout[...] = minp_v
        pltpu.sync_copy(vout, ominp.at[wid])

    return sc_kernel


def _softplus(x):
    u = jax.lax.exp2(jnp.abs(x) * jnp.float32(-1.4426950408889634))
    return jnp.maximum(x, 0.0) + jnp.log1p(u)


def _fixup_block(pred_ref, target_ref, out_ref):
    x = pred_ref[...]
    y = target_ref[...]
    s = _softplus(x)
    pos = jnp.sum(y, axis=1)
    d_f = jnp.maximum(_C - (_RATIO + 1.0) * pos, 0.0)

    @pl.when(pl.program_id(0) == 0)
    def _init():
        out_ref[...] = jnp.zeros((1, 1), jnp.float32)

    b = jax.lax.bitcast_convert_type(x, jnp.int32)
    ikey = b ^ ((b >> 31) & jnp.int32(0x7FFFFFFF))
    ikey = jnp.where(y > 0.5, jnp.int32(0x7FFFFFFF), ikey)
    d = d_f.astype(jnp.int32)
    cnt_neg = jnp.sum((ikey < 0).astype(jnp.int32), axis=1)
    t0 = jnp.where(cnt_neg >= d, jnp.int32(-2147483648), jnp.int32(0))

    def body(j, t):
        cand = t + (jnp.int32(1) << (30 - j))
        cnt = jnp.sum((ikey < cand[:, None]).astype(jnp.int32), axis=1)
        return jnp.where(cnt < d, cand, t)

    t = jax.lax.fori_loop(0, 31, body, t0)
    below = ikey < t[:, None]
    cnt_lt = jnp.sum(below.astype(jnp.int32), axis=1)
    sum_below = jnp.sum(jnp.where(below, s, 0.0), axis=1)
    bv = t ^ ((t >> 31) & jnp.int32(0x7FFFFFFF))
    sv = _softplus(jax.lax.bitcast_convert_type(bv, jnp.float32))
    corr = sum_below + (d - cnt_lt).astype(jnp.float32) * sv
    corr = jnp.where(d > 0, corr, 0.0)
    out_ref[...] += jnp.sum(corr).reshape(1, 1)


def _fixup_call(pred, target):
    n, c = pred.shape
    r = 512
    out = pl.pallas_call(
        _fixup_block,
        grid=(n // r,),
        in_specs=[
            pl.BlockSpec((r, c), lambda i: (i, 0)),
            pl.BlockSpec((r, c), lambda i: (i, 0)),
        ],
        out_specs=pl.BlockSpec((1, 1), lambda i: (0, 0)),
        out_shape=jax.ShapeDtypeStruct((1, 1), jnp.float32),
    )(pred, target)
    return out[0, 0]


def _block_kernel(pred_ref, target_ref, num_ref, den_ref, acc_ref):
    x = pred_ref[...]
    y = target_ref[...]
    s = _softplus(x)
    contrib = s - y * x
    i = pl.program_id(0)

    @pl.when(i == 0)
    def _init():
        num_ref[...] = jnp.zeros((1, 1), jnp.float32)
        den_ref[...] = jnp.zeros((1, 1), jnp.float32)
        acc_ref[...] = jnp.zeros_like(acc_ref)

    acc_ref[...] += contrib
    pos = jnp.sum(y, axis=1)
    den_ref[...] += jnp.sum(pos).reshape(1, 1)
    d_f = jnp.maximum(_C - (_RATIO + 1.0) * pos, 0.0)

    @pl.when(jnp.any(d_f > 0.0))
    def _rare_correction():
        b = jax.lax.bitcast_convert_type(x, jnp.int32)
        ikey = b ^ ((b >> 31) & jnp.int32(0x7FFFFFFF))
        ikey = jnp.where(y > 0.5, jnp.int32(0x7FFFFFFF), ikey)
        d = d_f.astype(jnp.int32)
        cnt_neg = jnp.sum((ikey < 0).astype(jnp.int32), axis=1)
        t0 = jnp.where(cnt_neg >= d, jnp.int32(-2147483648), jnp.int32(0))

        def body(j, t):
            cand = t + (jnp.int32(1) << (30 - j))
            cnt = jnp.sum((ikey < cand[:, None]).astype(jnp.int32), axis=1)
            return jnp.where(cnt < d, cand, t)

        t = jax.lax.fori_loop(0, 31, body, t0)
        below = ikey < t[:, None]
        cnt_lt = jnp.sum(below.astype(jnp.int32), axis=1)
        sum_below = jnp.sum(jnp.where(below, s, 0.0), axis=1)
        bv = t ^ ((t >> 31) & jnp.int32(0x7FFFFFFF))
        sv = _softplus(jax.lax.bitcast_convert_type(bv, jnp.float32))
        corr = sum_below + (d - cnt_lt).astype(jnp.float32) * sv
        corr = jnp.where(d > 0, corr, 0.0)
        num_ref[...] += -jnp.sum(corr).reshape(1, 1)

    @pl.when(i == pl.num_programs(0) - 1)
    def _finalize():
        num_ref[...] += jnp.sum(acc_ref[...]).reshape(1, 1)


_TC_ROWS = 13312
_R = 512


def kernel(pred, target):
    n, c = pred.shape
    sc = _make_sc_call(n - _TC_ROWS, _TC_ROWS)
    onum, oden, ominp = sc(pred, target)

    num_tc, den_tc = pl.pallas_call(
        _block_kernel,
        grid=(_TC_ROWS // _R,),
        in_specs=[
            pl.BlockSpec((_R, c), lambda i: (i, 0)),
            pl.BlockSpec((_R, c), lambda i: (i, 0)),
        ],
        out_specs=[
            pl.BlockSpec((1, 1), lambda i: (0, 0)),
            pl.BlockSpec((1, 1), lambda i: (0, 0)),
        ],
        out_shape=[
            jax.ShapeDtypeStruct((1, 1), jnp.float32),
            jax.ShapeDtypeStruct((1, 1), jnp.float32),
        ],
        scratch_shapes=[pltpu.VMEM((_R, c), jnp.float32)],
    )(pred, target)

    num_sc = jnp.sum(onum)
    den_sc = jnp.sum(oden[:, 0])
    minp = jnp.min(ominp)
    corr = jax.lax.cond(
        minp * (_RATIO + 1.0) < _C,
        lambda: _fixup_call(pred[_TC_ROWS:], target[_TC_ROWS:]),
        lambda: jnp.float32(0.0),
    )
    num = num_tc[0, 0] + num_sc - corr
    den = den_tc[0, 0] + den_sc
    return (num / c) / den


# speedup vs baseline: 1.0246x; 1.0246x over previous
import functools

import jax
import jax.numpy as jnp
from jax import lax
from jax.experimental import pallas as pl
from jax.experimental.pallas import tpu as pltpu
from jax.experimental.pallas import tpu_sc as plsc

_C = 1000
_RATIO = 3
_G_COEF = (0.9999999208709724, -0.4999925589798402, 0.333160106694737,
           -0.24825648584296323, 0.1905595564947489, -0.13583941890268855,
           0.07754038118242754, -0.029210267904388937, 0.00518600370045478)
_NW = 32
_SC_CH = 16


def _sc_contrib(xv, yv):
    u = jnp.exp(-jnp.abs(xv))
    g = jnp.float32(_G_COEF[8])
    for k in range(7, -1, -1):
        g = g * u + jnp.float32(_G_COEF[k])
    s = jnp.maximum(xv, 0.0) + u * g
    return s - yv * xv


def _make_sc_call(n_rows, base_offset):
    rpw = n_rows // _NW
    nchunks = rpw // _SC_CH
    mesh = plsc.VectorSubcoreMesh(core_axis_name="c", subcore_axis_name="s")

    @functools.partial(
        pl.kernel,
        mesh=mesh,
        out_type=[
            jax.ShapeDtypeStruct((_NW, 16), jnp.float32),
            jax.ShapeDtypeStruct((_NW, 16), jnp.float32),
            jax.ShapeDtypeStruct((_NW, 16), jnp.float32),
        ],
        scratch_types=[
            pltpu.VMEM((2, _SC_CH, _C), jnp.float32),
            pltpu.VMEM((2, _SC_CH, _C), jnp.float32),
            pltpu.VMEM((16,), jnp.float32),
            pltpu.SemaphoreType.DMA((2,)),
            pltpu.SemaphoreType.DMA((2,)),
        ],
        compiler_params=pltpu.CompilerParams(needs_layout_passes=False),
    )
    def sc_kernel(pred_hbm, target_hbm, onum, oden, ominp,
                  bufx, bufy, vout, semx, semy):
        wid = lax.axis_index("s") * 2 + lax.axis_index("c")
        base = base_offset + wid * rpw
        tailmask = lax.iota(jnp.int32, 16) < 8

        def issue(chunk, slot):
            row0 = base + chunk * _SC_CH
            pltpu.make_async_copy(
                pred_hbm.at[pl.ds(row0, _SC_CH), :], bufx.at[slot], semx.at[slot]
            ).start()
            pltpu.make_async_copy(
                target_hbm.at[pl.ds(row0, _SC_CH), :], bufy.at[slot], semy.at[slot]
            ).start()

        issue(0, 0)

        def chunk_body(ci, carry):
            accnum, den_v, minp_v = carry
            slot = lax.rem(ci, 2)

            @pl.when(ci + 1 < nchunks)
            def _issue_next():
                issue(ci + 1, 1 - slot)

            row0 = base + ci * _SC_CH
            pltpu.make_async_copy(
                pred_hbm.at[pl.ds(row0, _SC_CH), :], bufx.at[slot], semx.at[slot]
            ).wait()
            pltpu.make_async_copy(
                target_hbm.at[pl.ds(row0, _SC_CH), :], bufy.at[slot], semy.at[slot]
            ).wait()

            def row_body(r, carry2):
                accs, dv, mv = carry2
                accs = list(accs)
                poss = [jnp.zeros((16,), jnp.int32) for _ in range(4)]
                for k in range(62):
                    xv = bufx[slot, r, pl.ds(k * 16, 16)]
                    yv = bufy[slot, r, pl.ds(k * 16, 16)]
                    accs[k % 8] = accs[k % 8] + _sc_contrib(xv, yv)
                    poss[k % 4] = poss[k % 4] + plsc.all_reduce_population_count(yv > 0.5)
                xv = bufx[slot, r, pl.ds(984, 16)]
                yv = bufy[slot, r, pl.ds(984, 16)]
                xv = jnp.where(tailmask, jnp.float32(-1e30), xv)
                yv = jnp.where(tailmask, jnp.float32(0.0), yv)
                accs[6] = accs[6] + _sc_contrib(xv, yv)
                poss[2] = poss[2] + plsc.all_reduce_population_count(yv > 0.5)
                pos = (poss[0] + poss[1]) + (poss[2] + poss[3])
                posf = pos.astype(jnp.float32)
                return (tuple(accs), dv + posf, jnp.minimum(mv, posf))

            return lax.fori_loop(0, _SC_CH, row_body, (accnum, den_v, minp_v))

        acc0 = tuple(jnp.zeros((16,), jnp.float32) for _ in range(8))
        accs, den_v, minp_v = lax.fori_loop(
            0, nchunks, chunk_body,
            (acc0, jnp.zeros((16,), jnp.float32), jnp.full((16,), 1e9, jnp.float32)),
        )
        a01 = (accs[0] + accs[1]) + (accs[2] + accs[3])
        a23 = (accs[4] + accs[5]) + (accs[6] + accs[7])
        vout[...] = a01 + a23
        pltpu.sync_copy(vout, onum.at[wid])
        vout[...] = den_v
        pltpu.sync_copy(vout, oden.at[wid])
        vout[...] = minp_v
        pltpu.sync_copy(vout, ominp.at[wid])

    return sc_kernel


def _softplus(x):
    u = jax.lax.exp2(jnp.abs(x) * jnp.float32(-1.4426950408889634))
    return jnp.maximum(x, 0.0) + jnp.log1p(u)


def _fixup_block(pred_ref, target_ref, out_ref):
    x = pred_ref[...]
    y = target_ref[...]
    s = _softplus(x)
    pos = jnp.sum(y, axis=1)
    d_f = jnp.maximum(_C - (_RATIO + 1.0) * pos, 0.0)

    @pl.when(pl.program_id(0) == 0)
    def _init():
        out_ref[...] = jnp.zeros((1, 1), jnp.float32)

    b = jax.lax.bitcast_convert_type(x, jnp.int32)
    ikey = b ^ ((b >> 31) & jnp.int32(0x7FFFFFFF))
    ikey = jnp.where(y > 0.5, jnp.int32(0x7FFFFFFF), ikey)
    d = d_f.astype(jnp.int32)
    cnt_neg = jnp.sum((ikey < 0).astype(jnp.int32), axis=1)
    t0 = jnp.where(cnt_neg >= d, jnp.int32(-2147483648), jnp.int32(0))

    def body(j, t):
        cand = t + (jnp.int32(1) << (30 - j))
        cnt = jnp.sum((ikey < cand[:, None]).astype(jnp.int32), axis=1)
        return jnp.where(cnt < d, cand, t)

    t = jax.lax.fori_loop(0, 31, body, t0)
    below = ikey < t[:, None]
    cnt_lt = jnp.sum(below.astype(jnp.int32), axis=1)
    sum_below = jnp.sum(jnp.where(below, s, 0.0), axis=1)
    bv = t ^ ((t >> 31) & jnp.int32(0x7FFFFFFF))
    sv = _softplus(jax.lax.bitcast_convert_type(bv, jnp.float32))
    corr = sum_below + (d - cnt_lt).astype(jnp.float32) * sv
    corr = jnp.where(d > 0, corr, 0.0)
    out_ref[...] += jnp.sum(corr).reshape(1, 1)


def _fixup_call(pred, target):
    n, c = pred.shape
    r = 512
    out = pl.pallas_call(
        _fixup_block,
        grid=(n // r,),
        in_specs=[
            pl.BlockSpec((r, c), lambda i: (i, 0)),
            pl.BlockSpec((r, c), lambda i: (i, 0)),
        ],
        out_specs=pl.BlockSpec((1, 1), lambda i: (0, 0)),
        out_shape=jax.ShapeDtypeStruct((1, 1), jnp.float32),
    )(pred, target)
    return out[0, 0]


def _block_kernel(pred_ref, target_ref, num_ref, den_ref, acc_ref):
    x = pred_ref[...]
    y = target_ref[...]
    s = _softplus(x)
    contrib = s - y * x
    i = pl.program_id(0)

    @pl.when(i == 0)
    def _init():
        num_ref[...] = jnp.zeros((1, 1), jnp.float32)
        den_ref[...] = jnp.zeros((1, 1), jnp.float32)
        acc_ref[...] = jnp.zeros_like(acc_ref)

    acc_ref[...] += contrib
    pos = jnp.sum(y, axis=1)
    den_ref[...] += jnp.sum(pos).reshape(1, 1)
    d_f = jnp.maximum(_C - (_RATIO + 1.0) * pos, 0.0)

    @pl.when(jnp.any(d_f > 0.0))
    def _rare_correction():
        b = jax.lax.bitcast_convert_type(x, jnp.int32)
        ikey = b ^ ((b >> 31) & jnp.int32(0x7FFFFFFF))
        ikey = jnp.where(y > 0.5, jnp.int32(0x7FFFFFFF), ikey)
        d = d_f.astype(jnp.int32)
        cnt_neg = jnp.sum((ikey < 0).astype(jnp.int32), axis=1)
        t0 = jnp.where(cnt_neg >= d, jnp.int32(-2147483648), jnp.int32(0))

        def body(j, t):
            cand = t + (jnp.int32(1) << (30 - j))
            cnt = jnp.sum((ikey < cand[:, None]).astype(jnp.int32), axis=1)
            return jnp.where(cnt < d, cand, t)

        t = jax.lax.fori_loop(0, 31, body, t0)
        below = ikey < t[:, None]
        cnt_lt = jnp.sum(below.astype(jnp.int32), axis=1)
        sum_below = jnp.sum(jnp.where(below, s, 0.0), axis=1)
        bv = t ^ ((t >> 31) & jnp.int32(0x7FFFFFFF))
        sv = _softplus(jax.lax.bitcast_convert_type(bv, jnp.float32))
        corr = sum_below + (d - cnt_lt).astype(jnp.float32) * sv
        corr = jnp.where(d > 0, corr, 0.0)
        num_ref[...] += -jnp.sum(corr).reshape(1, 1)

    @pl.when(i == pl.num_programs(0) - 1)
    def _finalize():
        num_ref[...] += jnp.sum(acc_ref[...]).reshape(1, 1)


_TC_ROWS = 11776
_R = 512


def kernel(pred, target):
    n, c = pred.shape
    sc = _make_sc_call(n - _TC_ROWS, _TC_ROWS)
    onum, oden, ominp = sc(pred, target)

    num_tc, den_tc = pl.pallas_call(
        _block_kernel,
        grid=(_TC_ROWS // _R,),
        in_specs=[
            pl.BlockSpec((_R, c), lambda i: (i, 0)),
            pl.BlockSpec((_R, c), lambda i: (i, 0)),
        ],
        out_specs=[
            pl.BlockSpec((1, 1), lambda i: (0, 0)),
            pl.BlockSpec((1, 1), lambda i: (0, 0)),
        ],
        out_shape=[
            jax.ShapeDtypeStruct((1, 1), jnp.float32),
            jax.ShapeDtypeStruct((1, 1), jnp.float32),
        ],
        scratch_shapes=[pltpu.VMEM((_R, c), jnp.float32)],
    )(pred, target)

    num_sc = jnp.sum(onum)
    den_sc = jnp.sum(oden[:, 0])
    minp = jnp.min(ominp)
    corr = jax.lax.cond(
        minp * (_RATIO + 1.0) < _C,
        lambda: _fixup_call(pred[_TC_ROWS:], target[_TC_ROWS:]),
        lambda: jnp.float32(0.0),
    )
    num = num_tc[0, 0] + num_sc - corr
    den = den_tc[0, 0] + den_sc
    return (num / c) / den
